# baseline (speedup 1.0000x reference)
import functools
import math

import jax
import jax.numpy as jnp
from jax import lax
from jax.experimental import pallas as pl
from jax.experimental.pallas import tpu as pltpu
from jax.experimental.pallas import tpu_sc as plsc

MODEL_DIM = 64
LANES = 16
NUM_CORES = 2
NUM_SUBCORES = 16
NW = NUM_CORES * NUM_SUBCORES
BL = 128
DSUB = 8
NBUF = 4
HALF = 2
SCALE = 8.0


def _make_emb_kernel(S: int, T: int, D: int):
    assert S == NW * BL and D == MODEL_DIM
    n_dt = D // DSUB
    assert T % NBUF == 0 and T // NBUF >= 2

    mesh = plsc.VectorSubcoreMesh(core_axis_name="c", subcore_axis_name="s")
    DH = D // 2

    @functools.partial(
        pl.kernel,
        mesh=mesh,
        out_type=jax.ShapeDtypeStruct((T, n_dt, NW, DSUB * BL), jnp.float32),
        compiler_params=pltpu.CompilerParams(use_tc_tiling_on_sc=False,
                                             needs_layout_passes=False),
        scratch_types=[
            pltpu.VMEM((T, BL), jnp.int32),
            pltpu.VMEM((NBUF, 2, BL, DH), jnp.float32),
            pltpu.VMEM((NBUF, n_dt, DSUB * BL), jnp.float32),
            pltpu.SemaphoreType.DMA((NBUF,)),
            pltpu.SemaphoreType.DMA((NBUF,)),
        ],
    )
    def emb(tl_hbm, tr_hbm, idx_hbm, out_hbm, idx_v, rows_v, tr_v, gsem,
            wsem):
        wid = lax.axis_index("s") * NUM_CORES + lax.axis_index("c")
        pltpu.sync_copy(idx_hbm.at[wid], idx_v)

        iota = jnp.arange(LANES, dtype=jnp.int32)

        def issue_gather(t, b):
            pltpu.async_copy(tl_hbm.at[idx_v.at[t]], rows_v.at[b, 0],
                             gsem.at[b])
            pltpu.async_copy(tr_hbm.at[idx_v.at[t]], rows_v.at[b, 1],
                             gsem.at[b])

        def wait_gather(t, b):
            pltpu.make_async_copy(tl_hbm.at[idx_v.at[t]], rows_v.at[b, 0],
                                  gsem.at[b]).wait()
            pltpu.make_async_copy(tr_hbm.at[idx_v.at[t]], rows_v.at[b, 1],
                                  gsem.at[b]).wait()

        def wait_writeback(b):
            for dt in range(n_dt):
                pltpu.make_async_copy(tr_v.at[b, dt], out_hbm.at[0, dt, wid],
                                      wsem.at[b]).wait()

        def process(t, b):
            wait_gather(t, b)
            trb = tr_v.at[b]

            @plsc.parallel_loop(0, BL, step=LANES)
            def _(r0):
                ridx = iota + r0

                @plsc.parallel_loop(0, LANES, unroll=8)
                def _(j):
                    mj = (iota + j) % LANES
                    t0 = mj // DSUB
                    t1 = (mj % DSUB) * BL + ridx
                    for d0 in range(0, D, LANES):
                        block = rows_v.at[b, d0 // DH]
                        v = plsc.load_gather(block,
                                             [ridx, mj + d0 % DH])
                        plsc.store_scatter(trb, [t0 + d0 // DSUB, t1],
                                           v * SCALE)

            for dt in range(n_dt):
                pltpu.async_copy(trb.at[dt], out_hbm.at[t, dt, wid],
                                 wsem.at[b])

        for q in range(HALF):
            issue_gather(q, q)

        for b in range(NBUF):
            q = b + HALF
            if q >= NBUF:
                wait_writeback(q % NBUF)
            issue_gather(q, q % NBUF)
            process(b, b)

        def outer(go, carry):
            g0 = go * NBUF
            for b in range(NBUF):
                qb = (b + HALF) % NBUF
                wait_writeback(qb)
                issue_gather(g0 + b + HALF, qb)
                process(g0 + b, b)
            return carry

        lax.fori_loop(1, T // NBUF - 1, outer, 0)

        gl = T - NBUF
        for b in range(NBUF):
            if b < NBUF - HALF:
                qb = (b + HALF) % NBUF
                wait_writeback(qb)
                issue_gather(gl + b + HALF, qb)
            process(gl + b, b)

        for b in range(NBUF):
            wait_writeback(b)

    return emb


def kernel(x, table):
    S, T = x.shape
    D = table.shape[1]
    xw = x.astype(jnp.int32).reshape(NW, BL, T).transpose(0, 2, 1)
    out5 = _make_emb_kernel(S, T, D)(table[:, :D // 2], table[:, D // 2:], xw)
    y = out5.reshape(T, D // DSUB, NW, DSUB, BL)
    return y.transpose(2, 4, 0, 1, 3).reshape(S, T, D)


# speedup vs baseline: 2.0186x; 2.0186x over previous
import functools
import math

import jax
import jax.numpy as jnp
from jax import lax
from jax.experimental import pallas as pl
from jax.experimental.pallas import tpu as pltpu
from jax.experimental.pallas import tpu_sc as plsc

MODEL_DIM = 64
LANES = 16
NUM_CORES = 2
NUM_SUBCORES = 16
NW = NUM_CORES * NUM_SUBCORES
BL = 128
DSUB = 8
NBUF = 4
HALF = 2
SCALE = 8.0


def _make_emb_kernel(S: int, T: int, D: int):
    assert S == NW * BL and D == MODEL_DIM
    n_dt = D // DSUB
    assert T % NBUF == 0 and T // NBUF >= 2

    mesh = plsc.VectorSubcoreMesh(core_axis_name="c", subcore_axis_name="s")

    @functools.partial(
        pl.kernel,
        mesh=mesh,
        out_type=jax.ShapeDtypeStruct((T, n_dt, NW, DSUB * BL), jnp.float32),
        compiler_params=pltpu.CompilerParams(use_tc_tiling_on_sc=False,
                                             needs_layout_passes=False),
        scratch_types=[
            pltpu.VMEM((T, BL), jnp.int32),
            pltpu.VMEM((NBUF, BL, D), jnp.float32),
            pltpu.VMEM((NBUF, n_dt, DSUB * BL), jnp.float32),
            pltpu.SemaphoreType.DMA((NBUF,)),
            pltpu.SemaphoreType.DMA((NBUF,)),
        ],
    )
    def emb(table_hbm, idx_hbm, out_hbm, idx_v, rows_v, tr_v, gsem, wsem):
        wid = lax.axis_index("s") * NUM_CORES + lax.axis_index("c")
        pltpu.sync_copy(idx_hbm.at[wid], idx_v)

        iota = jnp.arange(LANES, dtype=jnp.int32)

        def issue_gather(t, b):
            pltpu.async_copy(table_hbm.at[idx_v.at[t]], rows_v.at[b],
                             gsem.at[b])

        def wait_gather(t, b):
            pltpu.make_async_copy(table_hbm.at[idx_v.at[t]], rows_v.at[b],
                                  gsem.at[b]).wait()

        def wait_writeback(b):
            for dt in range(n_dt):
                pltpu.make_async_copy(tr_v.at[b, dt], out_hbm.at[0, dt, wid],
                                      wsem.at[b]).wait()

        def process(t, b):
            wait_gather(t, b)
            block = rows_v.at[b]
            trb = tr_v.at[b]

            @plsc.parallel_loop(0, BL, step=LANES)
            def _(r0):
                ridx = iota + r0

                @plsc.parallel_loop(0, LANES, unroll=8)
                def _(j):
                    mj = (iota + j) % LANES
                    t0 = mj // DSUB
                    t1 = (mj % DSUB) * BL + ridx
                    for d0 in range(0, D, LANES):
                        v = plsc.load_gather(block, [ridx, mj + d0])
                        plsc.store_scatter(trb, [t0 + d0 // DSUB, t1],
                                           v * SCALE)

            for dt in range(n_dt):
                pltpu.async_copy(trb.at[dt], out_hbm.at[t, dt, wid],
                                 wsem.at[b])

        for q in range(HALF):
            issue_gather(q, q)

        for b in range(NBUF):
            q = b + HALF
            if q >= NBUF:
                wait_writeback(q % NBUF)
            issue_gather(q, q % NBUF)
            process(b, b)

        def outer(go, carry):
            g0 = go * NBUF
            for b in range(NBUF):
                qb = (b + HALF) % NBUF
                wait_writeback(qb)
                issue_gather(g0 + b + HALF, qb)
                process(g0 + b, b)
            return carry

        lax.fori_loop(1, T // NBUF - 1, outer, 0)

        gl = T - NBUF
        for b in range(NBUF):
            if b < NBUF - HALF:
                qb = (b + HALF) % NBUF
                wait_writeback(qb)
                issue_gather(gl + b + HALF, qb)
            process(gl + b, b)

        for b in range(NBUF):
            wait_writeback(b)

    return emb


def kernel(x, table):
    S, T = x.shape
    D = table.shape[1]
    xw = x.astype(jnp.int32).reshape(NW, BL, T).transpose(0, 2, 1)
    out5 = _make_emb_kernel(S, T, D)(table, xw)
    y = out5.reshape(T, D // DSUB, NW, DSUB, BL)
    return y.transpose(2, 4, 0, 1, 3).reshape(S, T, D)
